# baseline (speedup 1.0000x reference)
import functools

import numpy as np
import jax
import jax.numpy as jnp
from jax import lax
from jax.experimental import pallas as pl
from jax.experimental.pallas import tpu as pltpu
from jax.experimental.pallas import tpu_sc as plsc

_H = 512
_OUT = 384
_W0 = 96
_WROWS = 320
_LO = 128
_SCALE = 1.25
_NTILES = 32
_RPT = _OUT // _NTILES
_BROWS = 16
_SIDE = _BROWS * _H
_TBLK = _RPT * _OUT
_PLANE = _OUT * _OUT
_NPOS = 2 * _TBLK // 16


def _bilinear_mat():
    inv = _H / (_H * _SCALE)
    o = np.arange(_LO, _LO + _OUT, dtype=np.float64)
    s = (o + 0.5) * inv - 0.5
    k = np.arange(_H, dtype=np.float64)
    w = np.maximum(0.0, 1.0 - np.abs(s[:, None] - k[None, :]))
    w = w / w.sum(1, keepdims=True)
    return w.astype(np.float32)


def _gather_maps():
    src = np.arange(_H)
    tgt = np.round(src.astype(np.float32) * np.float32(_SCALE)).astype(np.int64)
    r = tgt - _LO
    ok = (r >= 0) & (r < _OUT)
    idx = np.full((_OUT,), -1, np.int64)
    idx[r[ok]] = src[ok]
    return idx


def _rlo(w):
    return (4 * (_LO + _RPT * w)) // 5 - 1


def _flat_index_table():
    idx = _gather_maps()
    tab = np.zeros((_NTILES, 2 * _TBLK), np.int32)
    for w in range(_NTILES):
        rlo = _rlo(w)
        assert _W0 <= rlo and rlo + _RPT <= _W0 + _WROWS
        for s in range(2):
            base = s * _SIDE
            gate = base + (_BROWS - 1) * _H
            for r in range(_RPT):
                sy = idx[w * _RPT + r]
                for c in range(_OUT):
                    sx = idx[c]
                    p = s * _TBLK + r * _OUT + c
                    if sy >= 0 and sx >= 0:
                        lrow = sy - rlo
                        assert 0 <= lrow < _RPT, (w, r, sy, rlo)
                        tab[w, p] = base + lrow * _H + sx
                    else:
                        tab[w, p] = gate
    return tab.reshape(-1)


_R = _bilinear_mat()
_CFLAT = _flat_index_table()


def _tc_body(il_ref, ir_ref, r_ref, oil_ref, oir_ref):
    r = r_ref[...]
    rt = r_ref[...].T
    for x_ref, o_ref in ((il_ref, oil_ref), (ir_ref, oir_ref)):
        for p in range(3):
            t = jnp.dot(r, x_ref[p], preferred_element_type=jnp.float32)
            o_ref[p] = jnp.dot(t, rt, preferred_element_type=jnp.float32)


def _sc_flow(dl_hbm, dr_hbm, mfl_hbm, mfr_hbm, cflat_hbm,
             odl, odr, oml, omr,
             cidx_v, b0_v, b1_v, bm_v, o0_v, o1_v, om_v, sem):
    wid = lax.axis_index("s") * 2 + lax.axis_index("c")
    rlo = (4 * (_LO + _RPT * wid)) // 5 - 1 - _W0
    boff = pl.multiple_of(rlo * _H, 8)
    coff = pl.multiple_of(wid * (2 * _TBLK), 2 * _TBLK)
    idx_cp = pltpu.make_async_copy(
        cflat_hbm.at[pl.ds(coff, 2 * _TBLK)], cidx_v, sem)
    idx_cp.start()

    n = _RPT * _H
    copies = []
    for s, (d_hbm, mf_hbm) in enumerate(((dl_hbm, mfl_hbm), (dr_hbm, mfr_hbm))):
        sb = s * _SIDE
        copies += [
            pltpu.make_async_copy(d_hbm.at[pl.ds(boff, n)],
                                  b0_v.at[pl.ds(sb, n)], sem),
            pltpu.make_async_copy(d_hbm.at[pl.ds(_WROWS * _H + boff, n)],
                                  b1_v.at[pl.ds(sb, n)], sem),
            pltpu.make_async_copy(mf_hbm.at[pl.ds(boff, n)],
                                  bm_v.at[pl.ds(sb, n)], sem),
        ]
    for c in copies:
        c.start()

    zero16 = jnp.zeros((16,), jnp.float32)
    gate0 = (_BROWS - 1) * _H

    @plsc.parallel_loop(0, _H // 16, unroll=2)
    def zbody(k):
        for buf in (b0_v, b1_v, bm_v):
            buf[pl.ds(gate0 + k * 16, 16)] = zero16
            buf[pl.ds(_SIDE + gate0 + k * 16, 16)] = zero16

    idx_cp.wait()
    for c in copies:
        c.wait()

    @plsc.parallel_loop(0, _NPOS, unroll=4)
    def jbody(i):
        pos = i * 16
        idx16 = cidx_v[pl.ds(pos, 16)]
        mg = plsc.load_gather(bm_v, [idx16])
        d0 = plsc.load_gather(b0_v, [idx16])
        d1 = plsc.load_gather(b1_v, [idx16])
        om_v[pl.ds(pos, 16)] = mg
        sm = mg * jnp.float32(_SCALE)
        o0_v[pl.ds(pos, 16)] = d0 * sm
        o1_v[pl.ds(pos, 16)] = d1 * sm

    ooff = pl.multiple_of(wid * _TBLK, _TBLK)
    pltpu.sync_copy(o0_v.at[pl.ds(0, _TBLK)], odl.at[pl.ds(ooff, _TBLK)])
    pltpu.sync_copy(o1_v.at[pl.ds(0, _TBLK)],
                    odl.at[pl.ds(_PLANE + ooff, _TBLK)])
    pltpu.sync_copy(om_v.at[pl.ds(0, _TBLK)], oml.at[pl.ds(ooff, _TBLK)])
    pltpu.sync_copy(o0_v.at[pl.ds(_TBLK, _TBLK)],
                    odr.at[pl.ds(ooff, _TBLK)])
    pltpu.sync_copy(o1_v.at[pl.ds(_TBLK, _TBLK)],
                    odr.at[pl.ds(_PLANE + ooff, _TBLK)])
    pltpu.sync_copy(om_v.at[pl.ds(_TBLK, _TBLK)], omr.at[pl.ds(ooff, _TBLK)])


_sc_call = functools.partial(
    pl.kernel,
    mesh=plsc.VectorSubcoreMesh(core_axis_name="c", subcore_axis_name="s"),
    compiler_params=pltpu.CompilerParams(
        use_tc_tiling_on_sc=False, needs_layout_passes=False),
    out_type=(
        jax.ShapeDtypeStruct((2 * _PLANE,), jnp.float32),
        jax.ShapeDtypeStruct((2 * _PLANE,), jnp.float32),
        jax.ShapeDtypeStruct((_PLANE,), jnp.float32),
        jax.ShapeDtypeStruct((_PLANE,), jnp.float32),
    ),
    scratch_types=[
        pltpu.VMEM((2 * _TBLK,), jnp.int32),
        pltpu.VMEM((2 * _SIDE,), jnp.float32),
        pltpu.VMEM((2 * _SIDE,), jnp.float32),
        pltpu.VMEM((2 * _SIDE,), jnp.float32),
        pltpu.VMEM((2 * _TBLK,), jnp.float32),
        pltpu.VMEM((2 * _TBLK,), jnp.float32),
        pltpu.VMEM((2 * _TBLK,), jnp.float32),
        pltpu.SemaphoreType.DMA,
    ],
)(_sc_flow)


def kernel(img_left, img_right, dsp_left, dsp_right, mask_left, mask_right):
    oil, oir = pl.pallas_call(
        _tc_body,
        out_shape=(
            jax.ShapeDtypeStruct((3, _OUT, _OUT), jnp.float32),
            jax.ShapeDtypeStruct((3, _OUT, _OUT), jnp.float32),
        ),
    )(img_left, img_right, jnp.asarray(_R))

    win = slice(_W0, _W0 + _WROWS)
    odl, odr, oml, omr = _sc_call(
        dsp_left[:, win, :].reshape(-1), dsp_right[:, win, :].reshape(-1),
        mask_left[win].astype(jnp.float32).reshape(-1),
        mask_right[win].astype(jnp.float32).reshape(-1),
        jnp.asarray(_CFLAT),
    )

    return (oil, oir,
            odl.reshape(2, _OUT, _OUT), odr.reshape(2, _OUT, _OUT),
            oml.reshape(_OUT, _OUT), omr.reshape(_OUT, _OUT))
